# baseline (speedup 1.0000x reference)
import functools

import jax
import jax.numpy as jnp
from jax import lax
from jax.experimental import pallas as pl
from jax.experimental.pallas import tpu as pltpu
from jax.experimental.pallas import tpu_sc as plsc

_ROWS = 10000
_COLS = 10000
_B = 16384

_NC = 2
_NS = 16
_NW = _NC * _NS
_BPW = _B // _NW
_CHUNK = 128
_NCHUNK = _BPW // _CHUNK
_L = 16


def _sc_gather(indices, flat_table):
    mesh = plsc.VectorSubcoreMesh(core_axis_name="c", subcore_axis_name="s")

    @functools.partial(
        pl.kernel,
        mesh=mesh,
        out_type=jax.ShapeDtypeStruct((_B,), jnp.float32),
        scratch_types=[
            pltpu.VMEM((_BPW,), jnp.int32),
            pltpu.VMEM((_BPW,), jnp.int32),
            pltpu.VMEM((_NCHUNK, _CHUNK), jnp.int32),
            pltpu.VMEM((_BPW,), jnp.float32),
            pltpu.SemaphoreType.DMA,
        ],
    )
    def k(idx_hbm, tab_hbm, out_hbm, row_v, col_v, flat_v, val_v, sem):
        wid = lax.axis_index("s") * _NC + lax.axis_index("c")
        base = wid * _BPW
        pltpu.sync_copy(idx_hbm.at[0, pl.ds(base, _BPW)], row_v)
        pltpu.sync_copy(idx_hbm.at[1, pl.ds(base, _BPW)], col_v)
        for j in range(_NCHUNK):
            for q in range(_CHUNK // _L):
                off = j * _CHUNK + q * _L
                r = row_v[pl.ds(off, _L)]
                c = col_v[pl.ds(off, _L)]
                flat_v[j, pl.ds(q * _L, _L)] = r * _COLS + c
        copies = [
            pltpu.async_copy(
                tab_hbm.at[flat_v.at[j]],
                val_v.at[pl.ds(j * _CHUNK, _CHUNK)],
                sem,
            )
            for j in range(_NCHUNK)
        ]
        for cp in copies:
            cp.wait()
        pltpu.sync_copy(val_v, out_hbm.at[pl.ds(base, _BPW)])

    return k(indices, flat_table)


def kernel(indices, table):
    idx = indices.astype(jnp.int32)
    flat = table.reshape(-1)
    return _sc_gather(idx, flat)


# speedup vs baseline: 5.8835x; 5.8835x over previous
import functools

import jax
import jax.numpy as jnp
from jax import lax
from jax.experimental import pallas as pl
from jax.experimental.pallas import tpu as pltpu
from jax.experimental.pallas import tpu_sc as plsc

_B = 16384

_NC = 2
_NS = 16
_NW = _NC * _NS
_BPW = _B // _NW
_CH = 16
_NCHUNK = _BPW // _CH


def _sc_gather(indices, table):
    mesh = plsc.VectorSubcoreMesh(core_axis_name="c", subcore_axis_name="s")

    @functools.partial(
        pl.kernel,
        mesh=mesh,
        out_type=jax.ShapeDtypeStruct((_B,), jnp.float32),
        scratch_types=[
            pltpu.VMEM((_BPW,), jnp.int32),
            pltpu.VMEM((_BPW,), jnp.int32),
            pltpu.VMEM((_CH * 8, 128), jnp.float32),
            pltpu.VMEM((_BPW,), jnp.float32),
            pltpu.SemaphoreType.DMA,
        ],
    )
    def k(idx_hbm, tab_hbm, out_hbm, row_v, col_v, blk_v, val_v, sem):
        wid = lax.axis_index("s") * _NC + lax.axis_index("c")
        base = wid * _BPW
        pltpu.sync_copy(idx_hbm.at[0, pl.ds(base, _BPW)], row_v)
        pltpu.sync_copy(idx_hbm.at[1, pl.ds(base, _BPW)], col_v)

        def chunk(g, carry):
            off = pl.multiple_of(g * _CH, _CH)
            rv = row_v[pl.ds(off, _CH)]
            cv = col_v[pl.ds(off, _CH)]
            for j in range(_CH):
                r0 = pl.multiple_of((rv[j] >> 3) * 8, 8)
                c0 = pl.multiple_of((cv[j] >> 7) * 128, 128)
                pltpu.make_async_copy(
                    tab_hbm.at[pl.ds(r0, 8), pl.ds(c0, 128)],
                    blk_v.at[pl.ds(j * 8, 8)],
                    sem,
                ).start()
            for j in range(_CH):
                pltpu.make_async_copy(
                    tab_hbm.at[pl.ds(0, 8), pl.ds(0, 128)],
                    blk_v.at[pl.ds(j * 8, 8)],
                    sem,
                ).wait()
            acc = jnp.zeros((_CH,), jnp.float32)
            pos = lax.iota(jnp.int32, _CH)
            lanes = cv & 15
            for j in range(_CH):
                sub = j * 8 + (rv[j] & 7)
                lane0 = pl.multiple_of(cv[j] & 112, 16)
                v = blk_v[sub, pl.ds(lane0, 16)]
                t = v.at[lanes].get(mode="promise_in_bounds")
                acc = jnp.where(pos == j, t, acc)
            val_v[pl.ds(off, _CH)] = acc
            return carry

        lax.fori_loop(0, _NCHUNK, chunk, 0)
        pltpu.sync_copy(val_v, out_hbm.at[pl.ds(base, _BPW)])

    return k(indices, table)


def kernel(indices, table):
    idx = indices.astype(jnp.int32)
    return _sc_gather(idx, table)


# speedup vs baseline: 7.9916x; 1.3583x over previous
import functools

import jax
import jax.numpy as jnp
from jax import lax
from jax.experimental import pallas as pl
from jax.experimental.pallas import tpu as pltpu
from jax.experimental.pallas import tpu_sc as plsc

_B = 16384

_NC = 2
_NS = 16
_NW = _NC * _NS
_BPW = _B // _NW
_CH = 16
_NCHUNK = _BPW // _CH


def _sc_gather(indices, table):
    mesh = plsc.VectorSubcoreMesh(core_axis_name="c", subcore_axis_name="s")

    @functools.partial(
        pl.kernel,
        mesh=mesh,
        out_type=jax.ShapeDtypeStruct((_B,), jnp.float32),
        scratch_types=[
            pltpu.VMEM((_BPW,), jnp.int32),
            pltpu.VMEM((_BPW,), jnp.int32),
            pltpu.VMEM((_CH * 8, 128), jnp.float32),
            pltpu.VMEM((_CH * 8, 128), jnp.float32),
            pltpu.VMEM((_CH * 8, 128), jnp.float32),
            pltpu.VMEM((_CH * 8, 128), jnp.float32),
            pltpu.VMEM((_BPW,), jnp.float32),
            pltpu.SemaphoreType.DMA,
            pltpu.SemaphoreType.DMA,
            pltpu.SemaphoreType.DMA,
            pltpu.SemaphoreType.DMA,
        ],
    )
    def k(idx_hbm, tab_hbm, out_hbm, row_v, col_v, b0, b1, b2, b3, val_v,
          s0, s1, s2, s3):
        wid = lax.axis_index("s") * _NC + lax.axis_index("c")
        base = wid * _BPW
        pltpu.sync_copy(idx_hbm.at[0, pl.ds(base, _BPW)], row_v)
        pltpu.sync_copy(idx_hbm.at[1, pl.ds(base, _BPW)], col_v)

        def fire(ch, bank, sem):
            off = pl.multiple_of(ch * _CH, _CH)
            rv = row_v[pl.ds(off, _CH)]
            cv = col_v[pl.ds(off, _CH)]
            for j in range(_CH):
                r0 = pl.multiple_of((rv[j] >> 3) * 8, 8)
                c0 = pl.multiple_of((cv[j] >> 7) * 128, 128)
                pltpu.make_async_copy(
                    tab_hbm.at[pl.ds(r0, 8), pl.ds(c0, 128)],
                    bank.at[pl.ds(j * 8, 8)],
                    sem,
                ).start()

        def drain(bank, sem):
            for j in range(_CH):
                pltpu.make_async_copy(
                    tab_hbm.at[pl.ds(0, 8), pl.ds(0, 128)],
                    bank.at[pl.ds(j * 8, 8)],
                    sem,
                ).wait()

        def extract(ch, bank):
            off = pl.multiple_of(ch * _CH, _CH)
            rv = row_v[pl.ds(off, _CH)]
            cv = col_v[pl.ds(off, _CH)]
            acc = jnp.zeros((_CH,), jnp.float32)
            pos = lax.iota(jnp.int32, _CH)
            lanes = cv & 15
            for j in range(_CH):
                sub = j * 8 + (rv[j] & 7)
                lane0 = pl.multiple_of(cv[j] & 112, 16)
                v = bank[sub, pl.ds(lane0, 16)]
                t = v.at[lanes].get(mode="promise_in_bounds")
                acc = jnp.where(pos == j, t, acc)
            val_v[pl.ds(off, _CH)] = acc

        banks = (b0, b1, b2, b3)
        sems = (s0, s1, s2, s3)

        fire(0, b0, s0)
        fire(1, b1, s1)
        fire(2, b2, s2)

        def quad(q, carry):
            c = q * 4
            for p in range(4):
                nxt = c + p + 3
                if p >= 1:
                    @pl.when(nxt < _NCHUNK)
                    def _():
                        fire(nxt, banks[(p + 3) % 4], sems[(p + 3) % 4])
                else:
                    fire(nxt, banks[3], sems[3])
                drain(banks[p], sems[p])
                extract(c + p, banks[p])
            return carry

        lax.fori_loop(0, _NCHUNK // 4, quad, 0)
        pltpu.sync_copy(val_v, out_hbm.at[pl.ds(base, _BPW)])

    return k(indices, table)


def kernel(indices, table):
    idx = indices.astype(jnp.int32)
    return _sc_gather(idx, table)


# speedup vs baseline: 8.0590x; 1.0084x over previous
import functools

import jax
import jax.numpy as jnp
from jax import lax
from jax.experimental import pallas as pl
from jax.experimental.pallas import tpu as pltpu
from jax.experimental.pallas import tpu_sc as plsc

_B = 16384

_NC = 2
_NS = 16
_NW = _NC * _NS
_BPW = _B // _NW
_CH = 16
_NCHUNK = _BPW // _CH


def _sc_gather(indices, table):
    mesh = plsc.VectorSubcoreMesh(core_axis_name="c", subcore_axis_name="s")

    @functools.partial(
        pl.kernel,
        mesh=mesh,
        out_type=jax.ShapeDtypeStruct((_B,), jnp.float32),
        scratch_types=[
            pltpu.VMEM((_BPW,), jnp.int32),
            pltpu.VMEM((_BPW,), jnp.int32),
            pltpu.VMEM((_CH * 8, 128), jnp.float32),
            pltpu.VMEM((_CH * 8, 128), jnp.float32),
            pltpu.VMEM((_CH * 8, 128), jnp.float32),
            pltpu.VMEM((_CH * 8, 128), jnp.float32),
            pltpu.VMEM((_BPW,), jnp.float32),
            pltpu.SemaphoreType.DMA,
            pltpu.SemaphoreType.DMA,
            pltpu.SemaphoreType.DMA,
            pltpu.SemaphoreType.DMA,
        ],
    )
    def k(idx_hbm, tab_hbm, out_hbm, row_v, col_v, b0, b1, b2, b3, val_v,
          s0, s1, s2, s3):
        wid = lax.axis_index("s") * _NC + lax.axis_index("c")
        base = wid * _BPW
        pltpu.sync_copy(idx_hbm.at[0, pl.ds(base, _BPW)], row_v)
        pltpu.sync_copy(idx_hbm.at[1, pl.ds(base, _BPW)], col_v)

        def fire(ch, bank, sem):
            off = pl.multiple_of(ch * _CH, _CH)
            r0v = (row_v[pl.ds(off, _CH)] >> 3) * 8
            c0v = (col_v[pl.ds(off, _CH)] >> 7) * 128
            for j in range(_CH):
                r0 = pl.multiple_of(r0v[j], 8)
                c0 = pl.multiple_of(c0v[j], 128)
                pltpu.make_async_copy(
                    tab_hbm.at[pl.ds(r0, 8), pl.ds(c0, 128)],
                    bank.at[pl.ds(j * 8, 8)],
                    sem,
                ).start()

        def drain(bank, sem):
            pltpu.make_async_copy(
                tab_hbm.at[pl.ds(0, _CH * 8), pl.ds(0, 128)],
                bank,
                sem,
            ).wait()

        def extract(ch, bank):
            off = pl.multiple_of(ch * _CH, _CH)
            rv = row_v[pl.ds(off, _CH)]
            cv = col_v[pl.ds(off, _CH)]
            acc = jnp.zeros((_CH,), jnp.float32)
            pos = lax.iota(jnp.int32, _CH)
            lanes = cv & 15
            subv = pos * 8 + (rv & 7)
            l0v = cv & 112
            for j in range(_CH):
                v = bank[subv[j], pl.ds(pl.multiple_of(l0v[j], 16), 16)]
                t = v.at[lanes].get(mode="promise_in_bounds")
                acc = jnp.where(pos == j, t, acc)
            val_v[pl.ds(off, _CH)] = acc

        banks = (b0, b1, b2, b3)
        sems = (s0, s1, s2, s3)

        fire(0, b0, s0)
        fire(1, b1, s1)
        fire(2, b2, s2)

        def quad(q, carry):
            c = q * 4
            for p in range(4):
                nxt = c + p + 3
                if p >= 1:
                    @pl.when(nxt < _NCHUNK)
                    def _():
                        fire(nxt, banks[(p + 3) % 4], sems[(p + 3) % 4])
                else:
                    fire(nxt, banks[3], sems[3])
                drain(banks[p], sems[p])
                extract(c + p, banks[p])
            return carry

        lax.fori_loop(0, _NCHUNK // 4, quad, 0)
        pltpu.sync_copy(val_v, out_hbm.at[pl.ds(base, _BPW)])

    return k(indices, table)


def kernel(indices, table):
    idx = indices.astype(jnp.int32)
    return _sc_gather(idx, table)
